# baseline (speedup 1.0000x reference)
import functools

import jax
import jax.numpy as jnp
from jax import lax
from jax.experimental import pallas as pl
from jax.experimental.pallas import tpu as pltpu
from jax.experimental.pallas import tpu_sc as plsc

N = 10000
E = 320000
D = 128
NC = 2
NS = 16
NW = NC * NS
B = 128
CPW = 79
PH = 48
EPW = E // NW
TRASH = 10016
DB = 80
DCPW = E // NW // DB
NPAD = NS * 640
RPS = 640
ZR = 128

_MESH = plsc.VectorSubcoreMesh(core_axis_name="c", subcore_axis_name="s")


@functools.partial(
    pl.kernel,
    out_type=jax.ShapeDtypeStruct((NC * NPAD,), jnp.float32),
    mesh=_MESH,
    scratch_types=[
        pltpu.VMEM((DCPW, DB), jnp.int32),
        pltpu.VMEM((DB,), jnp.float32),
        pltpu.VMEM((DB,), jnp.float32),
        pltpu.VMEM_SHARED((NPAD,), jnp.float32),
        pltpu.SemaphoreType.DMA,
    ],
)
def _deg_call(dst_hbm, out_hbm, dstv, onesv, zerosv, acc, sem):
    c = lax.axis_index("c")
    s = lax.axis_index("s")
    w = s * NC + c
    for i in range(DB // 16):
        onesv[pl.ds(i * 16, 16)] = jnp.ones((16,), jnp.float32)
        zerosv[pl.ds(i * 16, 16)] = jnp.zeros((16,), jnp.float32)
    for k in range(640 // DB):
        pltpu.sync_copy(zerosv, acc.at[pl.ds(s * 640 + k * DB, DB)])
    plsc.subcore_barrier()
    pltpu.sync_copy(dst_hbm.at[w], dstv)

    def body(j, carry):
        pltpu.async_copy(onesv, acc.at[dstv.at[j]], sem, add=True)
        return carry

    lax.fori_loop(0, DCPW, body, 0)

    def drain(j, carry):
        pltpu.make_async_copy(onesv, acc.at[dstv.at[j]], sem).wait()
        return carry

    lax.fori_loop(0, DCPW, drain, 0)
    plsc.subcore_barrier()
    pltpu.sync_copy(
        acc.at[pl.ds(s * 640, 640)], out_hbm.at[pl.ds(c * NPAD + s * 640, 640)]
    )


@functools.partial(
    pl.kernel,
    out_type=jax.ShapeDtypeStruct((NC, NPAD, D), jnp.float32),
    mesh=_MESH,
    scratch_types=[
        pltpu.VMEM((PH, B), jnp.int32),
        pltpu.VMEM((PH, B), jnp.int32),
        pltpu.VMEM((B, D), jnp.float32),
        pltpu.VMEM((B, D), jnp.float32),
        pltpu.VMEM_SHARED((NPAD, D), jnp.float32),
        pltpu.SemaphoreType.DMA,
        pltpu.SemaphoreType.DMA,
    ],
)
def _agg_call(g_hbm, src_hbm, dst_hbm, out_hbm, srcv, dstv, ra, rb, acc, g0, g1):
    c = lax.axis_index("c")
    s = lax.axis_index("s")
    w = s * NC + c
    rows = (ra, rb)
    gsem = (g0, g1)
    zv = jnp.zeros((16,), jnp.float32)

    def zbody(i, carry):
        for j in range(D // 16):
            ra[i, pl.ds(j * 16, 16)] = zv
        return carry

    lax.fori_loop(0, B, zbody, 0)
    for k in range(RPS // B):
        pltpu.sync_copy(ra, acc.at[pl.ds(s * RPS + k * B, B)])
    plsc.subcore_barrier()

    def gissue(b, j):
        pltpu.async_copy(g_hbm.at[srcv.at[j]], rows[b], gsem[b])

    def gwait(b, j):
        pltpu.make_async_copy(g_hbm.at[srcv.at[j]], rows[b], gsem[b]).wait()

    def step(b, j, j2):
        gwait(b, j)
        pltpu.sync_copy(rows[b], acc.at[dstv.at[j]], add=True)
        if j2 is not None:
            gissue(b, j2)

    def run_phase(np_chunks):
        gissue(0, 0)
        gissue(1, 1)
        ntrip = (np_chunks - 2) // 2

        def body(i, carry):
            j = 2 * i
            step(0, j, j + 2)
            step(1, j + 1, j + 3)
            return carry

        lax.fori_loop(0, ntrip - 1, body, 0)
        for jj in range(2 * (ntrip - 1), np_chunks):
            step(jj % 2, jj, jj + 2 if jj + 2 < np_chunks else None)

    pltpu.sync_copy(src_hbm.at[w, pl.ds(0, PH)], srcv)
    pltpu.sync_copy(dst_hbm.at[w, pl.ds(0, PH)], dstv)
    run_phase(PH)
    nb = CPW - PH
    pltpu.sync_copy(src_hbm.at[w, pl.ds(PH, nb)], srcv.at[pl.ds(0, nb)])
    pltpu.sync_copy(dst_hbm.at[w, pl.ds(PH, nb)], dstv.at[pl.ds(0, nb)])
    run_phase(nb)
    plsc.subcore_barrier()
    for k in range(RPS // ZR):
        base = s * RPS + k * ZR
        pltpu.sync_copy(acc.at[pl.ds(base, ZR)], out_hbm.at[c, pl.ds(base, ZR)])


BM = 1000


def _prep_body(dega_ref, degb_ref, x_ref, w1_ref, dinv_ref, g_ref):
    deg = dega_ref[...] + degb_ref[...]
    dinv = jnp.where(deg > 0, 1.0 / jnp.sqrt(jnp.maximum(deg, 1.0)), 0.0)
    dinv_ref[...] = dinv
    g_ref[...] = (
        jnp.dot(x_ref[...], w1_ref[...], preferred_element_type=jnp.float32) * dinv
    )


def _prep(dega, degb, x, w1):
    return pl.pallas_call(
        _prep_body,
        grid=(N // BM,),
        in_specs=[
            pl.BlockSpec((BM, 1), lambda i: (i, 0)),
            pl.BlockSpec((BM, 1), lambda i: (i, 0)),
            pl.BlockSpec((BM, D), lambda i: (i, 0)),
            pl.BlockSpec((D, D), lambda i: (0, 0)),
        ],
        out_specs=[
            pl.BlockSpec((BM, 1), lambda i: (i, 0)),
            pl.BlockSpec((BM, D), lambda i: (i, 0)),
        ],
        out_shape=[
            jax.ShapeDtypeStruct((N, 1), jnp.float32),
            jax.ShapeDtypeStruct((N, D), jnp.float32),
        ],
    )(dega, degb, x, w1)


def _mid_body(s1_ref, dinv_ref, b1_ref, w2_ref, h1_ref, g2_ref):
    stot = s1_ref[0] + s1_ref[1]
    h1 = jnp.maximum(stot * dinv_ref[...] + b1_ref[...], 0.0)
    h1_ref[...] = h1
    g2_ref[...] = (
        jnp.dot(h1, w2_ref[...], preferred_element_type=jnp.float32) * dinv_ref[...]
    )


def _mid(s1, dinv, b1, w2):
    return pl.pallas_call(
        _mid_body,
        grid=(N // BM,),
        in_specs=[
            pl.BlockSpec((NC, BM, D), lambda i: (0, i, 0)),
            pl.BlockSpec((BM, 1), lambda i: (i, 0)),
            pl.BlockSpec((1, D), lambda i: (0, 0)),
            pl.BlockSpec((D, D), lambda i: (0, 0)),
        ],
        out_specs=[
            pl.BlockSpec((BM, D), lambda i: (i, 0)),
            pl.BlockSpec((BM, D), lambda i: (i, 0)),
        ],
        out_shape=[
            jax.ShapeDtypeStruct((N, D), jnp.float32),
            jax.ShapeDtypeStruct((N, D), jnp.float32),
        ],
    )(s1, dinv, b1, w2)


def _out_body(s2_ref, dinv_ref, b2_ref, h2_ref):
    h2_ref[...] = (s2_ref[0] + s2_ref[1]) * dinv_ref[...] + b2_ref[...]


def _out(s2, dinv, b2):
    return pl.pallas_call(
        _out_body,
        grid=(N // BM,),
        in_specs=[
            pl.BlockSpec((NC, BM, D), lambda i: (0, i, 0)),
            pl.BlockSpec((BM, 1), lambda i: (i, 0)),
            pl.BlockSpec((1, D), lambda i: (0, 0)),
        ],
        out_specs=pl.BlockSpec((BM, D), lambda i: (i, 0)),
        out_shape=jax.ShapeDtypeStruct((N, D), jnp.float32),
    )(s2, dinv, b2)


def kernel(x, edge_index, W1, b1, W2, b2):
    pad = CPW * B - EPW
    src3 = jnp.pad(
        edge_index[0].reshape(NW, EPW), ((0, 0), (0, pad)), constant_values=0
    ).reshape(NW, CPW, B)
    dst3 = jnp.pad(
        edge_index[1].reshape(NW, EPW), ((0, 0), (0, pad)), constant_values=TRASH
    ).reshape(NW, CPW, B)
    dst3d = edge_index[1].reshape(NW, DCPW, DB)
    deg1d = _deg_call(dst3d)
    dega = deg1d[:N].reshape(N, 1)
    degb = deg1d[NPAD : NPAD + N].reshape(N, 1)
    dinv, g1 = _prep(dega, degb, x, W1)
    s1 = _agg_call(g1, src3, dst3)
    h1, g2 = _mid(s1, dinv, b1.reshape(1, D), W2)
    s2 = _agg_call(g2, src3, dst3)
    h2 = _out(s2, dinv, b2.reshape(1, D))
    return (h1, h2)


# speedup vs baseline: 1.9965x; 1.9965x over previous
import functools

import jax
import jax.numpy as jnp
from jax import lax
from jax.experimental import pallas as pl
from jax.experimental.pallas import tpu as pltpu
from jax.experimental.pallas import tpu_sc as plsc

N = 10000
E = 320000
D = 128
NC = 2
NS = 16
NW = NC * NS
B = 80
CPW = 125
PH = 64
DB = 80
DCPW = E // NW // DB
NPAD = NS * 640
RPS = 640
ZR = 128

_MESH = plsc.VectorSubcoreMesh(core_axis_name="c", subcore_axis_name="s")


@functools.partial(
    pl.kernel,
    out_type=jax.ShapeDtypeStruct((NC * NPAD,), jnp.float32),
    mesh=_MESH,
    scratch_types=[
        pltpu.VMEM((DCPW, DB), jnp.int32),
        pltpu.VMEM((DB,), jnp.float32),
        pltpu.VMEM((DB,), jnp.float32),
        pltpu.VMEM_SHARED((NPAD,), jnp.float32),
        pltpu.SemaphoreType.DMA,
    ],
)
def _deg_call(dst_hbm, out_hbm, dstv, onesv, zerosv, acc, sem):
    c = lax.axis_index("c")
    s = lax.axis_index("s")
    w = s * NC + c
    for i in range(DB // 16):
        onesv[pl.ds(i * 16, 16)] = jnp.ones((16,), jnp.float32)
        zerosv[pl.ds(i * 16, 16)] = jnp.zeros((16,), jnp.float32)
    for k in range(640 // DB):
        pltpu.sync_copy(zerosv, acc.at[pl.ds(s * 640 + k * DB, DB)])
    plsc.subcore_barrier()
    pltpu.sync_copy(dst_hbm.at[w], dstv)

    def body(j, carry):
        pltpu.async_copy(onesv, acc.at[dstv.at[j]], sem, add=True)
        return carry

    lax.fori_loop(0, DCPW, body, 0)

    def drain(j, carry):
        pltpu.make_async_copy(onesv, acc.at[dstv.at[j]], sem).wait()
        return carry

    lax.fori_loop(0, DCPW, drain, 0)
    plsc.subcore_barrier()
    pltpu.sync_copy(
        acc.at[pl.ds(s * 640, 640)], out_hbm.at[pl.ds(c * NPAD + s * 640, 640)]
    )


@functools.partial(
    pl.kernel,
    out_type=jax.ShapeDtypeStruct((NC, NPAD, D), jnp.float32),
    mesh=_MESH,
    scratch_types=[
        pltpu.VMEM((PH, B), jnp.int32),
        pltpu.VMEM((PH, B), jnp.int32),
        pltpu.VMEM((B, D), jnp.float32),
        pltpu.VMEM((B, D), jnp.float32),
        pltpu.VMEM((B, D), jnp.float32),
        pltpu.VMEM_SHARED((NPAD, D), jnp.float32),
        pltpu.SemaphoreType.DMA,
        pltpu.SemaphoreType.DMA,
        pltpu.SemaphoreType.DMA,
    ],
)
def _agg_call(g_hbm, src_hbm, dst_hbm, out_hbm, srcv, dstv, ra, rb, rc, acc, g0, g1, g2):
    c = lax.axis_index("c")
    s = lax.axis_index("s")
    w = s * NC + c
    rows = (ra, rb, rc)
    gsem = (g0, g1, g2)
    zv = jnp.zeros((16,), jnp.float32)

    def zbody(i, carry):
        for j in range(D // 16):
            ra[i, pl.ds(j * 16, 16)] = zv
        return carry

    lax.fori_loop(0, B, zbody, 0)
    for k in range(RPS // B):
        pltpu.sync_copy(ra, acc.at[pl.ds(s * RPS + k * B, B)])
    plsc.subcore_barrier()

    def gissue(b, j):
        pltpu.async_copy(g_hbm.at[srcv.at[j]], rows[b], gsem[b])

    def gwait(b, j):
        pltpu.make_async_copy(g_hbm.at[srcv.at[j]], rows[b], gsem[b]).wait()

    def step(b, j, j2):
        gwait(b, j)
        if j2 is not None:
            gissue((b + 2) % 3, j2)
        pltpu.sync_copy(rows[b], acc.at[dstv.at[j]], add=True)

    def run_phase(np_chunks):
        gissue(0, 0)
        gissue(1, 1)
        ntrip = (np_chunks - 2) // 3

        def body(i, carry):
            for k in range(3):
                j = 3 * i + k
                step(k, j, j + 2)
            return carry

        lax.fori_loop(0, ntrip, body, 0)
        for j in range(3 * ntrip, np_chunks):
            step(j % 3, j, j + 2 if j + 2 < np_chunks else None)

    pltpu.sync_copy(src_hbm.at[w, pl.ds(0, PH)], srcv)
    pltpu.sync_copy(dst_hbm.at[w, pl.ds(0, PH)], dstv)
    run_phase(PH)
    nb = CPW - PH
    pltpu.sync_copy(src_hbm.at[w, pl.ds(PH, nb)], srcv.at[pl.ds(0, nb)])
    pltpu.sync_copy(dst_hbm.at[w, pl.ds(PH, nb)], dstv.at[pl.ds(0, nb)])
    run_phase(nb)
    plsc.subcore_barrier()
    for k in range(RPS // ZR):
        base = s * RPS + k * ZR
        pltpu.sync_copy(acc.at[pl.ds(base, ZR)], out_hbm.at[c, pl.ds(base, ZR)])


BM = 2000


def _prep_body(dega_ref, degb_ref, x_ref, w1_ref, dinv_ref, g_ref):
    deg = dega_ref[...] + degb_ref[...]
    dinv = jnp.where(deg > 0, 1.0 / jnp.sqrt(jnp.maximum(deg, 1.0)), 0.0)
    dinv_ref[...] = dinv
    g_ref[...] = (
        jnp.dot(x_ref[...], w1_ref[...], preferred_element_type=jnp.float32) * dinv
    )


def _prep(dega, degb, x, w1):
    return pl.pallas_call(
        _prep_body,
        grid=(N // BM,),
        in_specs=[
            pl.BlockSpec((BM, 1), lambda i: (i, 0)),
            pl.BlockSpec((BM, 1), lambda i: (i, 0)),
            pl.BlockSpec((BM, D), lambda i: (i, 0)),
            pl.BlockSpec((D, D), lambda i: (0, 0)),
        ],
        out_specs=[
            pl.BlockSpec((BM, 1), lambda i: (i, 0)),
            pl.BlockSpec((BM, D), lambda i: (i, 0)),
        ],
        out_shape=[
            jax.ShapeDtypeStruct((N, 1), jnp.float32),
            jax.ShapeDtypeStruct((N, D), jnp.float32),
        ],
    )(dega, degb, x, w1)


def _mid_body(s1_ref, dinv_ref, b1_ref, w2_ref, h1_ref, g2_ref):
    stot = s1_ref[0] + s1_ref[1]
    h1 = jnp.maximum(stot * dinv_ref[...] + b1_ref[...], 0.0)
    h1_ref[...] = h1
    g2_ref[...] = (
        jnp.dot(h1, w2_ref[...], preferred_element_type=jnp.float32) * dinv_ref[...]
    )


def _mid(s1, dinv, b1, w2):
    return pl.pallas_call(
        _mid_body,
        grid=(N // BM,),
        in_specs=[
            pl.BlockSpec((NC, BM, D), lambda i: (0, i, 0)),
            pl.BlockSpec((BM, 1), lambda i: (i, 0)),
            pl.BlockSpec((1, D), lambda i: (0, 0)),
            pl.BlockSpec((D, D), lambda i: (0, 0)),
        ],
        out_specs=[
            pl.BlockSpec((BM, D), lambda i: (i, 0)),
            pl.BlockSpec((BM, D), lambda i: (i, 0)),
        ],
        out_shape=[
            jax.ShapeDtypeStruct((N, D), jnp.float32),
            jax.ShapeDtypeStruct((N, D), jnp.float32),
        ],
    )(s1, dinv, b1, w2)


def _out_body(s2_ref, dinv_ref, b2_ref, h2_ref):
    h2_ref[...] = (s2_ref[0] + s2_ref[1]) * dinv_ref[...] + b2_ref[...]


def _out(s2, dinv, b2):
    return pl.pallas_call(
        _out_body,
        grid=(N // BM,),
        in_specs=[
            pl.BlockSpec((NC, BM, D), lambda i: (0, i, 0)),
            pl.BlockSpec((BM, 1), lambda i: (i, 0)),
            pl.BlockSpec((1, D), lambda i: (0, 0)),
        ],
        out_specs=pl.BlockSpec((BM, D), lambda i: (i, 0)),
        out_shape=jax.ShapeDtypeStruct((N, D), jnp.float32),
    )(s2, dinv, b2)


def kernel(x, edge_index, W1, b1, W2, b2):
    src3 = edge_index[0].reshape(NW, CPW, B)
    dst3 = edge_index[1].reshape(NW, CPW, B)
    deg1d = _deg_call(dst3)
    dega = deg1d[:N].reshape(N, 1)
    degb = deg1d[NPAD : NPAD + N].reshape(N, 1)
    dinv, g1 = _prep(dega, degb, x, W1)
    s1 = _agg_call(g1, src3, dst3)
    h1, g2 = _mid(s1, dinv, b1.reshape(1, D), W2)
    s2 = _agg_call(g2, src3, dst3)
    h2 = _out(s2, dinv, b2.reshape(1, D))
    return (h1, h2)
